# baseline (speedup 1.0000x reference)
import jax
import jax.numpy as jnp
from jax import lax
from jax.experimental import pallas as pl
from jax.experimental.pallas import tpu as pltpu
from jax.experimental.pallas import tpu_sc as plsc

B = 4096
S = 200
D = 64
NC = 2
NS = 16
NW = NC * NS
SEQ_PER_W = B // NW
L = 16


def _emb_body(ids_hbm, tok_hbm, pos_hbm, out_hbm, idx_v, rows_v, pos_v, sem):
    wid = lax.axis_index("s") * NC + lax.axis_index("c")
    pltpu.sync_copy(pos_hbm, pos_v)
    base = wid * (SEQ_PER_W * S)

    def chunk(k, carry):
        start = base + k * S
        pltpu.sync_copy(ids_hbm.at[pl.ds(start, S)], idx_v)
        pltpu.async_copy(tok_hbm.at[idx_v], rows_v, sem).wait()

        def add_row(r, c2):
            for c in range(D // L):
                sl = pl.ds(c * L, L)
                rows_v[r, sl] = rows_v[r, sl] + pos_v[r, sl]
            return c2

        lax.fori_loop(0, S, add_row, 0)
        pltpu.sync_copy(rows_v, out_hbm.at[pl.ds(start, S)])
        return carry

    lax.fori_loop(0, SEQ_PER_W, chunk, 0)


def kernel(input_ids, token_table, pos_table):
    flat_ids = input_ids.reshape(-1).astype(jnp.int32)
    mesh = plsc.VectorSubcoreMesh(core_axis_name="c", subcore_axis_name="s")
    f = pl.kernel(
        _emb_body,
        mesh=mesh,
        compiler_params=pltpu.CompilerParams(use_tc_tiling_on_sc=False),
        out_type=jax.ShapeDtypeStruct((B * S, D), jnp.float32),
        scratch_types=[
            pltpu.VMEM((S,), jnp.int32),
            pltpu.VMEM((S, D), jnp.float32),
            pltpu.VMEM((S, D), jnp.float32),
            pltpu.SemaphoreType.DMA,
        ],
    )
    out = f(flat_ids, token_table, pos_table)
    return out.reshape(B, S, D)


# speedup vs baseline: 1.2138x; 1.2138x over previous
import jax
import jax.numpy as jnp
from jax import lax
from jax.experimental import pallas as pl
from jax.experimental.pallas import tpu as pltpu
from jax.experimental.pallas import tpu_sc as plsc

B = 4096
S = 200
D = 64
NC = 2
NS = 16
NW = NC * NS
SEQ_PER_W = B // NW
L = 16
NBUF = 4


def _emb_body(ids_hbm, tok_hbm, pos_hbm, out_hbm,
              idx_v, rb0, rb1, rb2, rb3, pos_v,
              gs0, gs1, gs2, gs3, os0, os1, os2, os3):
    wid = lax.axis_index("s") * NC + lax.axis_index("c")
    base = wid * (SEQ_PER_W * S)
    pltpu.sync_copy(pos_hbm, pos_v)
    pltpu.sync_copy(ids_hbm.at[pl.ds(base, SEQ_PER_W * S)], idx_v)

    rows = (rb0, rb1, rb2, rb3)
    gsem = (gs0, gs1, gs2, gs3)
    osem = (os0, os1, os2, os3)

    def gather_copy(k, b):
        idx = idx_v.at[pl.ds(k * S, S)]
        return pltpu.make_async_copy(tok_hbm.at[idx], rows[b], gsem[b])

    def out_copy(k, b):
        return pltpu.make_async_copy(rows[b], out_hbm.at[pl.ds(base + k * S, S)], osem[b])

    def add_pos(b):
        r = rows[b]

        @plsc.parallel_loop(0, S, 1, unroll=4)
        def _(i):
            for c in range(D // L):
                sl = pl.ds(c * L, L)
                r[i, sl] = r[i, sl] + pos_v[i, sl]

    def step(k, b, wait_prev_out, prefetch):
        gather_copy(k, b).wait()
        add_pos(b)
        out_copy(k, b).start()
        bb = (b + 3) % NBUF
        if wait_prev_out:
            out_copy(k - 1, bb).wait()
        if prefetch:
            gather_copy(k + 3, bb).start()

    for k in range(3):
        gather_copy(k, k).start()
    step(0, 0, False, True)
    for b in range(1, NBUF):
        step(b, b, True, True)

    def group(g, carry):
        for b in range(NBUF):
            step(g * NBUF + b, b, True, True)
        return carry

    lax.fori_loop(1, SEQ_PER_W // NBUF - 1, group, 0)

    k0 = SEQ_PER_W - NBUF
    step(k0, 0, True, True)
    for b in range(1, NBUF):
        step(k0 + b, b, True, False)
    out_copy(SEQ_PER_W - 1, NBUF - 1).wait()


def kernel(input_ids, token_table, pos_table):
    flat_ids = input_ids.reshape(-1).astype(jnp.int32)
    mesh = plsc.VectorSubcoreMesh(core_axis_name="c", subcore_axis_name="s")
    f = pl.kernel(
        _emb_body,
        mesh=mesh,
        compiler_params=pltpu.CompilerParams(use_tc_tiling_on_sc=False),
        out_type=jax.ShapeDtypeStruct((B * S, D), jnp.float32),
        scratch_types=(
            [pltpu.VMEM((SEQ_PER_W * S,), jnp.int32)]
            + [pltpu.VMEM((S, D), jnp.float32) for _ in range(NBUF)]
            + [pltpu.VMEM((S, D), jnp.float32)]
            + [pltpu.SemaphoreType.DMA for _ in range(2 * NBUF)]
        ),
    )
    out = f(flat_ids, token_table, pos_table)
    return out.reshape(B, S, D)
